# baseline (speedup 1.0000x reference)
import functools

import jax
import jax.numpy as jnp
from jax import lax
from jax.experimental import pallas as pl
from jax.experimental.pallas import tpu as pltpu
from jax.experimental.pallas import tpu_sc as plsc

NF = 384
B = 32
HW = 1024
CH = 128
NCH = NF // CH

NWORK = 4
RPW = B // NWORK


def _make_gather():
    mesh = plsc.VectorSubcoreMesh(core_axis_name="c", subcore_axis_name="s", num_cores=1)

    @functools.partial(
        pl.kernel,
        mesh=mesh,
        out_type=jax.ShapeDtypeStruct((B, 2 * NF), jnp.float32),
        scratch_types=[
            pltpu.VMEM((RPW,), jnp.int32),
            pltpu.VMEM((RPW, 2 * NF), jnp.float32),
            pltpu.SemaphoreType.DMA,
        ],
    )
    def gather(table_hbm, idx_hbm, out_hbm, idx_v, rows_v, sem):
        wid = lax.axis_index("s")

        @pl.when(wid < NWORK)
        def _():
            base = wid * RPW
            pltpu.sync_copy(idx_hbm.at[pl.ds(base, RPW)], idx_v)
            pltpu.async_copy(table_hbm.at[idx_v], rows_v, sem).wait()
            pltpu.sync_copy(rows_v, out_hbm.at[pl.ds(base, RPW)])

    return gather


_gather = _make_gather()


NB = 8
BLK = NB * HW


def _affine_body(e_ref, x_ref, _buf_ref, o_ref):
    for k in range(NB):
        g = e_ref[k, :NF]
        b = e_ref[k, NF:]
        rows = pl.ds(k * HW, HW)
        o_ref[rows, :] = x_ref[rows, :] * g + b


B0 = 8


def _affine_body0(y_ref, e_ref, x_ref, o_ref):
    g = e_ref[0, 0, :NF]
    b = e_ref[0, 0, NF:]
    o_ref[...] = x_ref[...] * g + b


def kernel(x, y, embed_weight):
    y32 = y.astype(jnp.int32)
    e = _gather(embed_weight, y32)
    H, W = x.shape[2], x.shape[3]
    xf = x.transpose(0, 2, 3, 1).reshape(B * HW, NF)
    grid_spec0 = pltpu.PrefetchScalarGridSpec(
        num_scalar_prefetch=1,
        grid=(B0,),
        in_specs=[
            pl.BlockSpec((1, 1, 2 * NF), lambda i, yv: (yv[i], 0, 0)),
            pl.BlockSpec((HW, NF), lambda i, yv: (i, 0)),
        ],
        out_specs=pl.BlockSpec((HW, NF), lambda i, yv: (i, 0)),
    )
    out0 = pl.pallas_call(
        _affine_body0,
        grid_spec=grid_spec0,
        out_shape=jax.ShapeDtypeStruct((B * HW, NF), jnp.float32),
    )(y32, embed_weight.reshape(-1, 1, 2 * NF), xf)
    off = B0 // NB
    out = pl.pallas_call(
        _affine_body,
        grid=((B - B0) * HW // BLK,),
        in_specs=[
            pl.BlockSpec((NB, 2 * NF), lambda i: (i + off, 0)),
            pl.BlockSpec((BLK, NF), lambda i: (i + off, 0)),
            pl.BlockSpec(memory_space=pl.ANY),
        ],
        out_specs=pl.BlockSpec((BLK, NF), lambda i: (i + off, 0)),
        out_shape=jax.ShapeDtypeStruct((B * HW, NF), jnp.float32),
        input_output_aliases={2: 0},
    )(e, xf, out0)
    return out.reshape(B, H, W, NF).transpose(0, 3, 1, 2)


# speedup vs baseline: 1.1128x; 1.1128x over previous
import functools

import jax
import jax.numpy as jnp
from jax import lax
from jax.experimental import pallas as pl
from jax.experimental.pallas import tpu as pltpu
from jax.experimental.pallas import tpu_sc as plsc

NF = 384
B = 32
HW = 1024
NWORK = 4
RPW = B // NWORK


def _make_gather():
    mesh = plsc.VectorSubcoreMesh(core_axis_name="c", subcore_axis_name="s", num_cores=1)

    @functools.partial(
        pl.kernel,
        mesh=mesh,
        out_type=jax.ShapeDtypeStruct((B, 2 * NF), jnp.float32),
        scratch_types=[
            pltpu.VMEM((RPW,), jnp.int32),
            pltpu.VMEM((RPW, 2 * NF), jnp.float32),
            pltpu.SemaphoreType.DMA,
        ],
    )
    def gather(table_hbm, idx_hbm, out_hbm, idx_v, rows_v, sem):
        wid = lax.axis_index("s")

        @pl.when(wid < NWORK)
        def _():
            base = wid * RPW
            pltpu.sync_copy(idx_hbm.at[pl.ds(base, RPW)], idx_v)
            pltpu.async_copy(table_hbm.at[idx_v], rows_v, sem).wait()
            pltpu.sync_copy(rows_v, out_hbm.at[pl.ds(base, RPW)])

    return gather


_gather = _make_gather()


NB = 8
BLK = NB * HW


def _affine_body(e_ref, x_ref, o_ref):
    for k in range(NB):
        g = e_ref[k, :NF]
        b = e_ref[k, NF:]
        rows = pl.ds(k * HW, HW)
        o_ref[rows, :] = x_ref[rows, :] * g + b


def kernel(x, y, embed_weight):
    y32 = y.astype(jnp.int32)
    e = _gather(embed_weight, y32)
    H, W = x.shape[2], x.shape[3]
    xf = x.transpose(0, 2, 3, 1).reshape(B * HW, NF)
    out = pl.pallas_call(
        _affine_body,
        grid=(B * HW // BLK,),
        in_specs=[
            pl.BlockSpec((NB, 2 * NF), lambda i: (i, 0)),
            pl.BlockSpec((BLK, NF), lambda i: (i, 0)),
        ],
        out_specs=pl.BlockSpec((BLK, NF), lambda i: (i, 0)),
        out_shape=jax.ShapeDtypeStruct((B * HW, NF), jnp.float32),
    )(e, xf)
    return out.reshape(B, H, W, NF).transpose(0, 3, 1, 2)
